# Initial kernel scaffold; baseline (speedup 1.0000x reference)
import jax
import jax.numpy as jnp
from jax.experimental import pallas as pl


def kernel(score1, score2, homo12):
    raise NotImplementedError("write your pallas kernel here")


# speedup vs baseline: 13.3245x; 13.3245x over previous
import functools

import jax
import jax.numpy as jnp
import numpy as np
from jax import lax
from jax.experimental import pallas as pl
from jax.experimental.pallas import tpu as pltpu
from jax.experimental.pallas import tpu_sc as plsc

B = 4
H = W = 384
HW = H * W
K = 512
NWORK = 32
ROWS_PW = (B * H) // NWORK
PIX_PW = ROWS_PW * W
QUART = HW // 4
CAP = 512
S = 4 * CAP

_f32 = jnp.float32
_i32 = jnp.int32

@functools.cache
def _mesh():
    return plsc.VectorSubcoreMesh(core_axis_name="c", subcore_axis_name="s")


def _worker_id():
    return lax.axis_index("s") * 2 + lax.axis_index("c")


def _coord_body(xs_ref, ys_ref, i00r, i01r, i10r, i11r,
                w00r, w01r, w10r, w11r):
    xs = jnp.clip(xs_ref[0], -4.0, 388.0)
    ys = jnp.clip(ys_ref[0], -4.0, 388.0)
    x0 = jnp.floor(xs)
    y0 = jnp.floor(ys)
    fx = xs - x0
    fy = ys - y0
    vx0 = (x0 >= 0.0) & (x0 <= 383.0)
    vx1 = (x0 + 1.0 >= 0.0) & (x0 + 1.0 <= 383.0)
    vy0 = (y0 >= 0.0) & (y0 <= 383.0)
    vy1 = (y0 + 1.0 >= 0.0) & (y0 + 1.0 <= 383.0)
    off = pl.program_id(0) * HW
    xi0 = jnp.clip(x0.astype(_i32), 0, 383)
    xi1 = jnp.clip(x0.astype(_i32) + 1, 0, 383)
    rb0 = jnp.clip(y0.astype(_i32), 0, 383) * W + off
    rb1 = jnp.clip(y0.astype(_i32) + 1, 0, 383) * W + off
    i00r[0] = rb0 + xi0
    i01r[0] = rb0 + xi1
    i10r[0] = rb1 + xi0
    i11r[0] = rb1 + xi1
    w00r[0] = jnp.where(vx0 & vy0, (1.0 - fx) * (1.0 - fy), 0.0)
    w01r[0] = jnp.where(vx1 & vy0, fx * (1.0 - fy), 0.0)
    w10r[0] = jnp.where(vx0 & vy1, (1.0 - fx) * fy, 0.0)
    w11r[0] = jnp.where(vx1 & vy1, fx * fy, 0.0)


def _coord_call(xs, ys):
    blk = pl.BlockSpec((1, H, W), lambda b: (b, 0, 0))
    return pl.pallas_call(
        _coord_body,
        grid=(B,),
        in_specs=[blk, blk],
        out_specs=[blk] * 8,
        out_shape=[jax.ShapeDtypeStruct((B, H, W), _i32)] * 4
        + [jax.ShapeDtypeStruct((B, H, W), _f32)] * 4,
    )(xs, ys)


_WIN = 4608


def _warp_body(img_hbm, i00f, i01f, i10f, i11f, w00f, w01f, w10f, w11f,
               out_hbm, ib0, ib1, ib2, ib3, wb0, wb1, wb2, wb3,
               gb0, gb1, gb2, gb3, obuf, sem):
    w = _worker_id()
    ibufs = (ib0, ib1, ib2, ib3)
    wbufs = (wb0, wb1, wb2, wb3)
    gbufs = (gb0, gb1, gb2, gb3)
    for wnd in range(PIX_PW // _WIN):
        base = w * PIX_PW + wnd * _WIN
        for k, src in enumerate((i00f, i01f, i10f, i11f)):
            pltpu.sync_copy(src.at[pl.ds(base, _WIN)], ibufs[k])
        cps = [pltpu.async_copy(img_hbm.at[ibufs[k]], gbufs[k], sem)
               for k in range(4)]
        for k, src in enumerate((w00f, w01f, w10f, w11f)):
            pltpu.sync_copy(src.at[pl.ds(base, _WIN)], wbufs[k])
        for cp in cps:
            cp.wait()

        def combine(j, carry):
            sl = pl.ds(pl.multiple_of(j * 16, 16), 16)
            obuf[sl] = (gb0[sl] * wb0[sl] + gb1[sl] * wb1[sl]
                        + gb2[sl] * wb2[sl] + gb3[sl] * wb3[sl])
            return carry

        lax.fori_loop(0, _WIN // 16, combine, jnp.int32(0))
        pltpu.sync_copy(obuf, out_hbm.at[pl.ds(base, _WIN)])


def _warp_call(img2, idxw):
    return pl.kernel(
        _warp_body,
        out_type=jax.ShapeDtypeStruct((B * HW,), _f32),
        mesh=_mesh(),
        compiler_params=pltpu.CompilerParams(needs_layout_passes=False),
        scratch_types=(
            [pltpu.VMEM((_WIN,), _i32) for _ in range(4)]
            + [pltpu.VMEM((_WIN,), _f32) for _ in range(8)]
            + [pltpu.VMEM((_WIN,), _f32), pltpu.SemaphoreType.DMA]
        ),
    )(img2, *idxw)


def _pool5(x):
    z = jnp.zeros((H, 2), _f32)
    p = jnp.concatenate([z, x, z], axis=1)
    m = p[:, 0:W]
    for k in range(1, 5):
        m = jnp.maximum(m, p[:, k:k + W])
    z2 = jnp.zeros((2, W), _f32)
    p2 = jnp.concatenate([z2, m, z2], axis=0)
    m2 = p2[0:H, :]
    for k in range(1, 5):
        m2 = jnp.maximum(m2, p2[k:k + H, :])
    return m2


def _nms_body(s1_ref, w2_ref, nms1_ref, nms2_ref, tau_ref):
    ri = lax.broadcasted_iota(_i32, (H, W), 0)
    ci = lax.broadcasted_iota(_i32, (H, W), 1)
    bm = (ri >= 8) & (ri < H - 8) & (ci >= 8) & (ci < W - 8)
    for m, (in_ref, out_ref) in enumerate(((s1_ref, nms1_ref),
                                           (w2_ref, nms2_ref))):
        x = jnp.where(bm, in_ref[0], 0.0)
        pooled = _pool5(x)
        nms = jnp.where((x == pooled) & (x > 0.0), x, 0.0)
        out_ref[0] = nms
        bits = lax.bitcast_convert_type(nms, _i32)

        def bs_body(_, carry):
            lo, hi = carry
            mid = lo + (hi - lo + 1) // 2
            cnt = jnp.sum((bits >= mid).astype(_i32))
            go = cnt >= K
            return jnp.where(go, mid, lo), jnp.where(go, hi, mid - 1)

        lo, _hi = lax.fori_loop(0, 31, bs_body,
                                (jnp.int32(0), jnp.int32(2**30)))
        tau = lax.bitcast_convert_type(lo, _f32)
        tau_ref[0, m, :] = jnp.zeros((128,), _f32) + tau


def _nms_call(s1, w2):
    return pl.pallas_call(
        _nms_body,
        grid=(B,),
        in_specs=[pl.BlockSpec((1, H, W), lambda b: (b, 0, 0)),
                  pl.BlockSpec((1, H, W), lambda b: (b, 0, 0))],
        out_specs=[pl.BlockSpec((1, H, W), lambda b: (b, 0, 0)),
                   pl.BlockSpec((1, H, W), lambda b: (b, 0, 0)),
                   pl.BlockSpec((1, 2, 128), lambda b: (b, 0, 0))],
        out_shape=[jax.ShapeDtypeStruct((B, H, W), _f32),
                   jax.ShapeDtypeStruct((B, H, W), _f32),
                   jax.ShapeDtypeStruct((B, 2, 128), _f32)],
    )(s1, w2)


def _compact_body(nms_hbm, tau_hbm, cv_hbm, ci_hbm, sbuf, vbuf, ibuf, tbuf):
    w = _worker_id()
    row = w // 4
    q = w % 4
    pltpu.sync_copy(tau_hbm.at[row], tbuf)
    tau = tbuf[:]
    pltpu.sync_copy(nms_hbm.at[row, pl.ds(q * QUART, QUART)], sbuf)
    padv = jnp.zeros((16,), _f32) - 1.0
    padi = jnp.zeros((16,), _i32)
    for t in range(CAP // 16):
        vbuf[pl.ds(t * 16, 16)] = padv
        ibuf[pl.ds(t * 16, 16)] = padi
    qbase = q * QUART

    def chunk(i, cursor):
        off = pl.multiple_of(i * 16, 16)
        v = sbuf[pl.ds(off, 16)]
        m = v >= tau
        cnt = jnp.sum(m.astype(_i32))

        @pl.when(cnt > 0)
        def _():
            pos = plsc.cumsum(m.astype(_i32)) - 1 + cursor
            msk = m & (pos < CAP)
            posc = jnp.clip(pos, 0, CAP - 1)
            plsc.store_scatter(vbuf, [posc], v, mask=msk)
            iv = lax.iota(_i32, 16) + (qbase + i * 16)
            plsc.store_scatter(ibuf, [posc], iv, mask=msk)

        return cursor + cnt

    lax.fori_loop(0, QUART // 16, chunk, jnp.int32(0))
    pltpu.sync_copy(vbuf, cv_hbm.at[w])
    pltpu.sync_copy(ibuf, ci_hbm.at[w])


def _compact_call(nms_flat, tau16):
    return pl.kernel(
        _compact_body,
        out_type=(jax.ShapeDtypeStruct((NWORK, CAP), _f32),
                  jax.ShapeDtypeStruct((NWORK, CAP), _i32)),
        mesh=_mesh(),
        compiler_params=pltpu.CompilerParams(needs_layout_passes=False),
        scratch_types=[pltpu.VMEM((QUART,), _f32), pltpu.VMEM((CAP,), _f32),
                       pltpu.VMEM((CAP,), _i32), pltpu.VMEM((16,), _f32)],
    )(nms_flat, tau16)


def _rank_body(cv_ref, ci_ref, cvt_ref, cit_ref, tau_ref, kp_ref, sv_ref):
    CH = 256
    for m in range(2):
        v = cv_ref[m, 0, 0, :]
        ix = ci_ref[m, 0, 0, :]
        tau = tau_ref[0, m, 0]
        n_gt = jnp.sum((v > tau).astype(_f32))
        need = jnp.float32(K) - n_gt
        vj = v[None, :]
        ij = ix[None, :]

        tseq_row = jnp.zeros((1, S), _f32)
        for c in range(S // CH):
            vjc = cvt_ref[m, 0, c * CH:(c + 1) * CH, :]
            ijc = cit_ref[m, 0, c * CH:(c + 1) * CH, :]
            contrib = (vjc == tau) & (ijc < ij)
            tseq_row = tseq_row + jnp.sum(contrib.astype(_f32), axis=0,
                                          keepdims=True)
        sel = (v > tau) | ((v == tau) & (tseq_row[0] < need))
        sval = jnp.where(sel, v, 0.0)
        sv_ref[m, 0, 0, :] = sval if m == 0 else -sval

        if m == 0:
            kiota = lax.broadcasted_iota(_i32, (CH, K), 1)
            accy = jnp.zeros((K,), _f32)
            accx = jnp.zeros((K,), _f32)
            for c in range(S // CH):
                vi = cvt_ref[m, 0, c * CH:(c + 1) * CH, :]
                ii = cit_ref[m, 0, c * CH:(c + 1) * CH, :]
                gt = (vj > vi) | ((vj == vi) & (ij < ii))
                rank_c = jnp.sum(gt.astype(_f32), axis=1,
                                 keepdims=True).astype(_i32)
                tseq_c = jnp.sum(((vj == tau) & (ij < ii)).astype(_f32),
                                 axis=1, keepdims=True)
                sel_c = (vi > tau) | ((vi == tau) & (tseq_c < need))
                t7 = ii >> 7
                yq = (t7 * 21846) >> 16
                xq = (t7 - 3 * yq) * 128 + (ii & 127)
                oh = jnp.where((rank_c == kiota) & sel_c, 1.0, 0.0)
                accy = accy + jnp.sum(oh * yq.astype(_f32), axis=0)
                accx = accx + jnp.sum(oh * xq.astype(_f32), axis=0)
            kp_ref[0, 0, :] = accy.astype(_i32)
            kp_ref[0, 1, :] = accx.astype(_i32)


def _rank_call(cv, ci, cvt, cit, tau):
    return pl.pallas_call(
        _rank_body,
        grid=(B,),
        in_specs=[pl.BlockSpec((2, 1, 1, S), lambda b: (0, b, 0, 0)),
                  pl.BlockSpec((2, 1, 1, S), lambda b: (0, b, 0, 0)),
                  pl.BlockSpec((2, 1, S, 1), lambda b: (0, b, 0, 0)),
                  pl.BlockSpec((2, 1, S, 1), lambda b: (0, b, 0, 0)),
                  pl.BlockSpec((1, 2, 128), lambda b: (b, 0, 0))],
        out_specs=[pl.BlockSpec((1, 2, K), lambda b: (b, 0, 0)),
                   pl.BlockSpec((2, 1, 1, S), lambda b: (0, b, 0, 0))],
        out_shape=[jax.ShapeDtypeStruct((B, 2, K), _i32),
                   jax.ShapeDtypeStruct((2, B, 1, S), _f32)],
    )(cv, ci, cvt, cit, tau)


def _scatter_body(sv_hbm, si_hbm, zero_hbm, d_hbm, dbuf, v0b, v1b, i0b, i1b):
    w = _worker_id()
    b = w // 8
    base = (w % 8) * PIX_PW
    pltpu.sync_copy(zero_hbm, dbuf)
    pltpu.sync_copy(sv_hbm.at[b], v0b)
    pltpu.sync_copy(sv_hbm.at[4 + b], v1b)
    pltpu.sync_copy(si_hbm.at[b], i0b)
    pltpu.sync_copy(si_hbm.at[4 + b], i1b)

    def body(i, carry):
        off = pl.multiple_of(i * 16, 16)
        for vb, ib in ((v0b, i0b), (v1b, i1b)):
            val = vb[pl.ds(off, 16)]
            loc = ib[pl.ds(off, 16)] - base
            msk = (loc >= 0) & (loc < PIX_PW)
            locc = jnp.clip(loc, 0, PIX_PW - 1)
            plsc.addupdate_scatter(dbuf, [locc], val, mask=msk)
        return carry

    lax.fori_loop(0, S // 16, body, jnp.int32(0))
    pltpu.sync_copy(dbuf, d_hbm.at[pl.ds(w * PIX_PW, PIX_PW)])


def _scatter_call(sv8, si8, dzero):
    return pl.kernel(
        _scatter_body,
        out_type=jax.ShapeDtypeStruct((B * HW,), _f32),
        mesh=_mesh(),
        compiler_params=pltpu.CompilerParams(needs_layout_passes=False),
        scratch_types=[pltpu.VMEM((PIX_PW,), _f32), pltpu.VMEM((S,), _f32),
                       pltpu.VMEM((S,), _f32), pltpu.VMEM((S,), _i32),
                       pltpu.VMEM((S,), _i32)],
    )(sv8, si8, dzero)


_gax = np.arange(15, dtype=np.float32) - 7.0
_g1 = np.exp(-(_gax**2) / (2.0 * 0.5**2), dtype=np.float32)
_g1 = (_g1 / _g1.sum()).astype(np.float32)
_GW = [float(x) for x in _g1]


def _loss_body(d_ref, xs_ref, ys_ref, out_ref):
    x = d_ref[0]
    z = jnp.zeros((H, 7), _f32)
    p = jnp.concatenate([z, x, z], axis=1)
    a = _GW[0] * p[:, 0:W]
    for k in range(1, 15):
        a = a + _GW[k] * p[:, k:k + W]
    z2 = jnp.zeros((7, W), _f32)
    p2 = jnp.concatenate([z2, a, z2], axis=0)
    d2 = _GW[0] * p2[0:H, :]
    for k in range(1, 15):
        d2 = d2 + _GW[k] * p2[k:k + H, :]

    xs = xs_ref[0]
    ys = ys_ref[0]
    x0 = jnp.floor(xs)
    y0 = jnp.floor(ys)
    wx1 = xs - x0
    wy1 = ys - y0
    wx0 = 1.0 - wx1
    wy0 = 1.0 - wy1
    vx0 = (x0 >= 0.0) & (x0 <= 383.0)
    vx1 = (x0 + 1.0 >= 0.0) & (x0 + 1.0 <= 383.0)
    vy0 = (y0 >= 0.0) & (y0 <= 383.0)
    vy1 = (y0 + 1.0 >= 0.0) & (y0 + 1.0 <= 383.0)
    s = (jnp.where(vx0 & vy0, wx0 * wy0, 0.0)
         + jnp.where(vx1 & vy0, wx1 * wy0, 0.0)
         + jnp.where(vx0 & vy1, wx0 * wy1, 0.0)
         + jnp.where(vx1 & vy1, wx1 * wy1, 0.0))
    vis = jnp.where(s > 0.0, 1.0, 0.0)
    num = jnp.sum(d2 * d2 * vis)
    den = jnp.sum(vis)
    out_ref[0, 0, :] = jnp.zeros((128,), _f32) + num
    out_ref[0, 1, :] = jnp.zeros((128,), _f32) + den


def _loss_call(d, xs, ys):
    blk = pl.BlockSpec((1, H, W), lambda b: (b, 0, 0))
    return pl.pallas_call(
        _loss_body,
        grid=(B,),
        in_specs=[blk, blk, blk],
        out_specs=[pl.BlockSpec((1, 2, 128), lambda b: (b, 0, 0))],
        out_shape=[jax.ShapeDtypeStruct((B, 2, 128), _f32)],
    )(d, xs, ys)[0]


def kernel(score1, score2, homo12):
    s1 = score1.reshape(B, H, W)
    img2 = score2.reshape(B * HW)
    ysg, xsg = jnp.meshgrid(jnp.arange(H, dtype=_f32),
                            jnp.arange(W, dtype=_f32), indexing='ij')
    grid = jnp.stack([xsg.ravel(), ysg.ravel(),
                      jnp.ones(H * W, _f32)], axis=0)
    warped = jnp.einsum('bij,jn->bin', homo12, grid)
    xs = (warped[:, 0] / (warped[:, 2] + 1e-8)).reshape(B, H, W)
    ys = (warped[:, 1] / (warped[:, 2] + 1e-8)).reshape(B, H, W)

    idxw = [a.reshape(B * HW) for a in _coord_call(xs, ys)]
    w2 = _warp_call(img2, idxw).reshape(B, H, W)
    nms1, nms2, taublk = _nms_call(s1, w2)
    nms_flat = jnp.stack([nms1, nms2]).reshape(8, HW)
    tau16 = jnp.transpose(taublk, (1, 0, 2))[:, :, :16].reshape(8, 16)

    cv, ci = _compact_call(nms_flat, tau16)
    cv4 = cv.reshape(2, B, 1, S)
    ci4 = ci.reshape(2, B, 1, S)
    kp_t, sv = _rank_call(cv4, ci4, cv.reshape(2, B, S, 1),
                          ci.reshape(2, B, S, 1), taublk)

    d = _scatter_call(sv.reshape(8, S), ci.reshape(8, S),
                      jnp.zeros((PIX_PW,), _f32)).reshape(B, H, W)
    nd = _loss_call(d, xs, ys)
    loss = 100.0 * jnp.sum(nd[:, 0, 0]) / jnp.sum(nd[:, 1, 0])
    kp1 = jnp.transpose(kp_t, (0, 2, 1)).reshape(B, 1, K, 2)
    return (loss, kp1)
